# baseline (speedup 1.0000x reference)
import functools

import jax
import jax.numpy as jnp
from jax import lax
from jax.experimental import pallas as pl
from jax.experimental.pallas import tpu as pltpu
from jax.experimental.pallas import tpu_sc as plsc

NC = 2
NS = 16
LCH = 128


def _make_agg(n, d, n_pad, ch):
  rows_per_tile = n_pad // NS
  chp = ch // 2
  mesh = plsc.VectorSubcoreMesh(
      core_axis_name="c", subcore_axis_name="s",
      num_cores=NC, num_subcores=NS)

  @functools.partial(
      pl.kernel,
      out_type=jax.ShapeDtypeStruct((NC, n_pad, d), jnp.float32),
      mesh=mesh,
      scratch_types=[
          pltpu.VMEM((chp, LCH), jnp.int32),
          pltpu.VMEM((chp, LCH), jnp.int32),
          pltpu.VMEM((LCH, d), jnp.float32),
          pltpu.VMEM((LCH, d), jnp.float32),
          pltpu.VMEM_SHARED((n_pad, d), jnp.float32),
          pltpu.SemaphoreType.DMA,
          pltpu.SemaphoreType.DMA,
      ],
  )
  def agg_kernel(h_hbm, srcp_hbm, dstp_hbm, zeros_hbm, out_hbm,
                 src_idx, dst_idx, buf_a, buf_b, acc, sem_a, sem_b):
    c = lax.axis_index("c")
    s = lax.axis_index("s")
    wid = c * NS + s

    pltpu.sync_copy(zeros_hbm,
                    acc.at[pl.ds(s * rows_per_tile, rows_per_tile)])
    plsc.subcore_barrier()

    def gather(j, buf, sem):
      pltpu.async_copy(h_hbm.at[src_idx.at[j]], buf, sem)

    def wait(buf, sem):
      pltpu.make_async_copy(h_hbm.at[pl.ds(0, LCH)], buf, sem).wait()

    def scatter_add(j, buf):
      pltpu.sync_copy(buf, acc.at[dst_idx.at[j]], add=True)

    @pl.loop(0, 2)
    def _(p):
      pltpu.sync_copy(srcp_hbm.at[wid, pl.ds(p * chp, chp)], src_idx)
      pltpu.sync_copy(dstp_hbm.at[wid, pl.ds(p * chp, chp)], dst_idx)

      gather(0, buf_a, sem_a)

      @pl.loop(0, chp, step=2)
      def _(g):
        gather(g + 1, buf_b, sem_b)
        wait(buf_a, sem_a)
        scatter_add(g, buf_a)

        @pl.when(g + 2 < chp)
        def _():
          gather(g + 2, buf_a, sem_a)

        wait(buf_b, sem_b)
        scatter_add(g + 1, buf_b)

    plsc.subcore_barrier()
    pltpu.sync_copy(acc.at[pl.ds(s * rows_per_tile, rows_per_tile)],
                    out_hbm.at[c, pl.ds(s * rows_per_tile, rows_per_tile)])

  return agg_kernel


def _mlp1_body(x_ref, p_ref, w1_ref, b1_ref, w2_ref, b2_ref, o_ref,
               *, n_real, bm):
  z = x_ref[...] + p_ref[0] + p_ref[1]
  t = jnp.dot(z, w1_ref[...], preferred_element_type=jnp.float32)
  t = jnp.maximum(t + b1_ref[...], 0.0)
  h = jnp.dot(t, w2_ref[...], preferred_element_type=jnp.float32)
  h = jnp.maximum(h + b2_ref[...], 0.0)
  rows = pl.program_id(0) * bm + jax.lax.broadcasted_iota(
      jnp.int32, h.shape, 0)
  o_ref[...] = jnp.where(rows < n_real, h, 0.0)


def _mlp2_body(h_ref, q_ref, w1_ref, b1_ref, w2_ref, b2_ref,
               wc_ref, bc_ref, o_ref):
  z = h_ref[...] + q_ref[0] + q_ref[1]
  t = jnp.dot(z, w1_ref[...], preferred_element_type=jnp.float32)
  t = jnp.maximum(t + b1_ref[...], 0.0)
  h2 = jnp.dot(t, w2_ref[...], preferred_element_type=jnp.float32)
  h2 = jnp.maximum(h2 + b2_ref[...], 0.0)
  o = jnp.dot(h2, wc_ref[...], preferred_element_type=jnp.float32)
  o_ref[...] = o + bc_ref[...]


def _full_spec(shape):
  return pl.BlockSpec(shape, lambda i: (0,) * len(shape))


def _mlp1_call(x_ext, p, w1, b1, w2, b2, bm, n_real):
  n_pad, d = x_ext.shape
  h = w1.shape[1]
  grid = (-(-n_pad // bm),)
  return pl.pallas_call(
      functools.partial(_mlp1_body, n_real=n_real, bm=bm),
      grid=grid,
      in_specs=[
          pl.BlockSpec((bm, d), lambda i: (i, 0)),
          pl.BlockSpec((NC, bm, d), lambda i: (0, i, 0)),
          _full_spec(w1.shape),
          _full_spec((1, h)),
          _full_spec(w2.shape),
          _full_spec((1, h)),
      ],
      out_specs=pl.BlockSpec((bm, h), lambda i: (i, 0)),
      out_shape=jax.ShapeDtypeStruct((n_pad, h), jnp.float32),
  )(x_ext, p, w1, b1.reshape(1, -1), w2, b2.reshape(1, -1))


def _mlp2_call(hh, q, w1, b1, w2, b2, wc, bc, bm, n_real):
  d = hh.shape[1]
  n = n_real
  h = w1.shape[1]
  c = wc.shape[1]
  grid = (n // bm,)
  return pl.pallas_call(
      _mlp2_body,
      grid=grid,
      in_specs=[
          pl.BlockSpec((bm, d), lambda i: (i, 0)),
          pl.BlockSpec((NC, bm, d), lambda i: (0, i, 0)),
          _full_spec(w1.shape),
          _full_spec((1, h)),
          _full_spec(w2.shape),
          _full_spec((1, h)),
          _full_spec(wc.shape),
          _full_spec((1, c)),
      ],
      out_specs=pl.BlockSpec((bm, c), lambda i: (i, 0)),
      out_shape=jax.ShapeDtypeStruct((n, c), jnp.float32),
  )(hh, q, w1, b1.reshape(1, -1), w2, b2.reshape(1, -1),
    wc, bc.reshape(1, -1))


def kernel(x, edge_index, W11, b11, W12, b12, W21, b21, W22, b22, Wc, bc):
  n, d = x.shape
  e = edge_index.shape[1]
  nw = NC * NS
  ch = -(-e // (nw * LCH))
  ch = -(-ch // 4) * 4
  e_pad = nw * ch * LCH

  n_pad = -(-(n + 1) // (NS * 8)) * (NS * 8)

  ei = edge_index.astype(jnp.int32)
  e_tile = -(-e // nw)
  ei = jnp.pad(ei, ((0, 0), (0, nw * e_tile - e)))
  pad_per_tile = ch * LCH - e_tile
  pad_src = jnp.full((nw, pad_per_tile), n, jnp.int32)
  pad_dst = jnp.broadcast_to(
      jnp.arange(pad_per_tile, dtype=jnp.int32) * 523 % n,
      (nw, pad_per_tile))
  real_valid = jnp.arange(nw * e_tile, dtype=jnp.int32).reshape(nw, e_tile) < e
  src2 = jnp.where(real_valid, ei[0].reshape(nw, e_tile), n)
  dst2 = jnp.where(real_valid, ei[1].reshape(nw, e_tile), 0)
  srcp = jnp.concatenate([src2, pad_src], axis=1).reshape(nw, ch, LCH)
  dstp = jnp.concatenate([dst2, pad_dst], axis=1).reshape(nw, ch, LCH)
  zeros = jnp.zeros((n_pad // NS, d), jnp.float32)

  agg = _make_agg(n, d, n_pad, ch)
  bm = 2000

  x_ext = jnp.pad(x, ((0, n_pad - n), (0, 0)))
  p1 = agg(x_ext, srcp, dstp, zeros)
  h1 = _mlp1_call(x_ext, p1, W11, b11, W12, b12, bm, n)
  p2 = agg(h1, srcp, dstp, zeros)
  return _mlp2_call(h1, p2, W21, b21, W22, b22, Wc, bc, bm, n)


# speedup vs baseline: 2.9714x; 2.9714x over previous
import functools

import jax
import jax.numpy as jnp
from jax import lax
from jax.experimental import pallas as pl
from jax.experimental.pallas import tpu as pltpu
from jax.experimental.pallas import tpu_sc as plsc

NC = 2
NS = 16
LCH = 128


def _make_agg(n, d, n_pad, ch):
  rows_per_tile = n_pad // NS
  chp = ch // 2
  mesh = plsc.VectorSubcoreMesh(
      core_axis_name="c", subcore_axis_name="s",
      num_cores=NC, num_subcores=NS)

  @functools.partial(
      pl.kernel,
      out_type=jax.ShapeDtypeStruct((NC, n_pad, d), jnp.float32),
      mesh=mesh,
      scratch_types=[
          pltpu.VMEM((chp, LCH), jnp.int32),
          pltpu.VMEM((chp, LCH), jnp.int32),
          pltpu.VMEM((LCH, d), jnp.float32),
          pltpu.VMEM((LCH, d), jnp.float32),
          pltpu.VMEM_SHARED((n_pad, d), jnp.float32),
          pltpu.SemaphoreType.DMA,
          pltpu.SemaphoreType.DMA,
      ],
  )
  def agg_kernel(h_hbm, srcp_hbm, dstp_hbm, zeros_hbm, out_hbm,
                 src_idx, dst_idx, buf_a, buf_b, acc, sem_a, sem_b):
    c = lax.axis_index("c")
    s = lax.axis_index("s")
    wid = c * NS + s

    pltpu.sync_copy(zeros_hbm,
                    acc.at[pl.ds(s * rows_per_tile, rows_per_tile)])
    plsc.subcore_barrier()

    def gather(j, buf, sem):
      pltpu.async_copy(h_hbm.at[src_idx.at[j]], buf, sem)

    def wait(buf, sem):
      pltpu.make_async_copy(h_hbm.at[pl.ds(0, LCH)], buf, sem).wait()

    def scatter_add(j, buf):
      pltpu.sync_copy(buf, acc.at[dst_idx.at[j]], add=True)

    @pl.loop(0, 2)
    def _(p):
      pltpu.sync_copy(srcp_hbm.at[wid, pl.ds(p * chp, chp)], src_idx)
      pltpu.sync_copy(dstp_hbm.at[wid, pl.ds(p * chp, chp)], dst_idx)

      gather(0, buf_a, sem_a)

      @pl.loop(0, chp, step=2)
      def _(g):
        gather(g + 1, buf_b, sem_b)
        wait(buf_a, sem_a)
        scatter_add(g, buf_a)

        @pl.when(g + 2 < chp)
        def _():
          gather(g + 2, buf_a, sem_a)

        wait(buf_b, sem_b)
        scatter_add(g + 1, buf_b)

    plsc.subcore_barrier()
    pltpu.sync_copy(acc.at[pl.ds(s * rows_per_tile, rows_per_tile)],
                    out_hbm.at[c, pl.ds(s * rows_per_tile, rows_per_tile)])

  return agg_kernel


def _mlp1_body(x_ref, p_ref, w1_ref, b1_ref, w2_ref, b2_ref, o_ref,
               *, n_real, bm):
  z = x_ref[...] + p_ref[0] + p_ref[1]
  t = jnp.dot(z, w1_ref[...], preferred_element_type=jnp.float32)
  t = jnp.maximum(t + b1_ref[...], 0.0)
  h = jnp.dot(t, w2_ref[...], preferred_element_type=jnp.float32)
  h = jnp.maximum(h + b2_ref[...], 0.0)
  rows = pl.program_id(0) * bm + jax.lax.broadcasted_iota(
      jnp.int32, h.shape, 0)
  o_ref[...] = jnp.where(rows < n_real, h, 0.0)


def _mlp2_body(h_ref, q_ref, w1_ref, b1_ref, w2_ref, b2_ref,
               wc_ref, bc_ref, o_ref):
  z = h_ref[...] + q_ref[0] + q_ref[1]
  t = jnp.dot(z, w1_ref[...], preferred_element_type=jnp.float32)
  t = jnp.maximum(t + b1_ref[...], 0.0)
  h2 = jnp.dot(t, w2_ref[...], preferred_element_type=jnp.float32)
  h2 = jnp.maximum(h2 + b2_ref[...], 0.0)
  o = jnp.dot(h2, wc_ref[...], preferred_element_type=jnp.float32)
  o_ref[...] = o + bc_ref[...]


def _full_spec(shape):
  return pl.BlockSpec(shape, lambda i: (0,) * len(shape))


def _mlp1_call(x_ext, p, w1, b1, w2, b2, bm, n_real):
  n_pad, d = x_ext.shape
  h = w1.shape[1]
  grid = (-(-n_pad // bm),)
  return pl.pallas_call(
      functools.partial(_mlp1_body, n_real=n_real, bm=bm),
      grid=grid,
      in_specs=[
          pl.BlockSpec((bm, d), lambda i: (i, 0)),
          pl.BlockSpec((NC, bm, d), lambda i: (0, i, 0)),
          _full_spec(w1.shape),
          _full_spec((1, h)),
          _full_spec(w2.shape),
          _full_spec((1, h)),
      ],
      out_specs=pl.BlockSpec((bm, h), lambda i: (i, 0)),
      out_shape=jax.ShapeDtypeStruct((n_pad, h), jnp.float32),
  )(x_ext, p, w1, b1.reshape(1, -1), w2, b2.reshape(1, -1))


def _mlp2_call(hh, q, w1, b1, w2, b2, wc, bc, bm, n_real):
  d = hh.shape[1]
  n = n_real
  h = w1.shape[1]
  c = wc.shape[1]
  grid = (n // bm,)
  return pl.pallas_call(
      _mlp2_body,
      grid=grid,
      in_specs=[
          pl.BlockSpec((bm, d), lambda i: (i, 0)),
          pl.BlockSpec((NC, bm, d), lambda i: (0, i, 0)),
          _full_spec(w1.shape),
          _full_spec((1, h)),
          _full_spec(w2.shape),
          _full_spec((1, h)),
          _full_spec(wc.shape),
          _full_spec((1, c)),
      ],
      out_specs=pl.BlockSpec((bm, c), lambda i: (i, 0)),
      out_shape=jax.ShapeDtypeStruct((n, c), jnp.float32),
  )(hh, q, w1, b1.reshape(1, -1), w2, b2.reshape(1, -1),
    wc, bc.reshape(1, -1))


def kernel(x, edge_index, W11, b11, W12, b12, W21, b21, W22, b22, Wc, bc):
  n, d = x.shape
  e = edge_index.shape[1]
  nw = NC * NS
  ch = -(-e // (nw * LCH))
  ch = -(-ch // 4) * 4
  e_pad = nw * ch * LCH

  n_pad = -(-(n + 1) // (NS * 8)) * (NS * 8)

  ei = edge_index.astype(jnp.int32)
  e_tile = -(-e // nw)
  ei = jnp.pad(ei, ((0, 0), (0, nw * e_tile - e)))
  pad_per_tile = ch * LCH - e_tile
  zrows = n_pad - n
  tile_off = jnp.arange(nw, dtype=jnp.int32)[:, None] * 7
  pad_i = jnp.arange(pad_per_tile, dtype=jnp.int32)[None, :]
  pad_src = n + (pad_i + tile_off) % zrows
  pad_dst = (pad_i * 523 + tile_off * 331) % n
  real_valid = jnp.arange(nw * e_tile, dtype=jnp.int32).reshape(nw, e_tile) < e
  src2 = jnp.where(real_valid, ei[0].reshape(nw, e_tile),
                   n + jnp.arange(e_tile, dtype=jnp.int32)[None, :] % zrows)
  dst2 = jnp.where(real_valid, ei[1].reshape(nw, e_tile), 0)
  srcp = jnp.concatenate([src2, pad_src], axis=1).reshape(nw, ch, LCH)
  dstp = jnp.concatenate([dst2, pad_dst], axis=1).reshape(nw, ch, LCH)
  zeros = jnp.zeros((n_pad // NS, d), jnp.float32)

  agg = _make_agg(n, d, n_pad, ch)
  bm = 2000

  x_ext = jnp.pad(x, ((0, n_pad - n), (0, 0)))
  p1 = agg(x_ext, srcp, dstp, zeros)
  h1 = _mlp1_call(x_ext, p1, W11, b11, W12, b12, bm, n)
  p2 = agg(h1, srcp, dstp, zeros)
  return _mlp2_call(h1, p2, W21, b21, W22, b22, Wc, bc, bm, n)


# speedup vs baseline: 2.9786x; 1.0024x over previous
import functools

import jax
import jax.numpy as jnp
from jax import lax
from jax.experimental import pallas as pl
from jax.experimental.pallas import tpu as pltpu
from jax.experimental.pallas import tpu_sc as plsc

NC = 2
NS = 16
LCH = 128


def _make_agg(n, d, n_pad, ch):
  rows_per_tile = n_pad // NS
  chp = ch // 2
  mesh = plsc.VectorSubcoreMesh(
      core_axis_name="c", subcore_axis_name="s",
      num_cores=NC, num_subcores=NS)

  @functools.partial(
      pl.kernel,
      out_type=jax.ShapeDtypeStruct((NC, n_pad, d), jnp.float32),
      mesh=mesh,
      scratch_types=[
          pltpu.VMEM((chp, LCH), jnp.int32),
          pltpu.VMEM((chp, LCH), jnp.int32),
          pltpu.VMEM((LCH, d), jnp.float32),
          pltpu.VMEM((LCH, d), jnp.float32),
          pltpu.VMEM_SHARED((n_pad, d), jnp.float32),
          pltpu.SemaphoreType.DMA,
          pltpu.SemaphoreType.DMA,
      ],
  )
  def agg_kernel(h_hbm, srcp_hbm, dstp_hbm, zeros_hbm, out_hbm,
                 src_idx, dst_idx, buf_a, buf_b, acc, sem_a, sem_b):
    c = lax.axis_index("c")
    s = lax.axis_index("s")
    wid = c * NS + s

    pltpu.sync_copy(zeros_hbm,
                    acc.at[pl.ds(s * rows_per_tile, rows_per_tile)])
    plsc.subcore_barrier()

    def gather(j, buf, sem):
      pltpu.async_copy(h_hbm.at[src_idx.at[j]], buf, sem)

    def wait(buf, sem):
      pltpu.make_async_copy(h_hbm.at[pl.ds(0, LCH)], buf, sem).wait()

    def scatter_add(j, buf):
      pltpu.sync_copy(buf, acc.at[dst_idx.at[j]], add=True)

    @pl.loop(0, 2)
    def _(p):
      pltpu.sync_copy(srcp_hbm.at[wid, pl.ds(p * chp, chp)], src_idx)
      pltpu.sync_copy(dstp_hbm.at[wid, pl.ds(p * chp, chp)], dst_idx)

      gather(0, buf_a, sem_a)

      @pl.loop(0, chp, step=2)
      def _(g):
        gather(g + 1, buf_b, sem_b)
        wait(buf_a, sem_a)
        scatter_add(g, buf_a)

        @pl.when(g + 2 < chp)
        def _():
          gather(g + 2, buf_a, sem_a)

        wait(buf_b, sem_b)
        scatter_add(g + 1, buf_b)

    plsc.subcore_barrier()
    pltpu.sync_copy(acc.at[pl.ds(s * rows_per_tile, rows_per_tile)],
                    out_hbm.at[c, pl.ds(s * rows_per_tile, rows_per_tile)])

  return agg_kernel


def _mlp1_body(x_ref, p_ref, w1_ref, b1_ref, w2_ref, b2_ref, o_ref,
               *, n_real, bm):
  z = x_ref[...] + p_ref[0] + p_ref[1]
  t = jnp.dot(z, w1_ref[...], preferred_element_type=jnp.float32)
  t = jnp.maximum(t + b1_ref[...], 0.0)
  h = jnp.dot(t, w2_ref[...], preferred_element_type=jnp.float32)
  h = jnp.maximum(h + b2_ref[...], 0.0)
  rows = pl.program_id(0) * bm + jax.lax.broadcasted_iota(
      jnp.int32, h.shape, 0)
  o_ref[...] = jnp.where(rows < n_real, h, 0.0)


def _mlp2_body(h_ref, q_ref, w1_ref, b1_ref, w2_ref, b2_ref,
               wc_ref, bc_ref, o_ref):
  z = h_ref[...] + q_ref[0] + q_ref[1]
  t = jnp.dot(z, w1_ref[...], preferred_element_type=jnp.float32)
  t = jnp.maximum(t + b1_ref[...], 0.0)
  h2 = jnp.dot(t, w2_ref[...], preferred_element_type=jnp.float32)
  h2 = jnp.maximum(h2 + b2_ref[...], 0.0)
  o = jnp.dot(h2, wc_ref[...], preferred_element_type=jnp.float32)
  o_ref[...] = o + bc_ref[...]


def _full_spec(shape):
  return pl.BlockSpec(shape, lambda i: (0,) * len(shape))


def _mlp1_call(x_ext, p, w1, b1, w2, b2, bm, n_real):
  n_pad, d = x_ext.shape
  h = w1.shape[1]
  grid = (-(-n_pad // bm),)
  return pl.pallas_call(
      functools.partial(_mlp1_body, n_real=n_real, bm=bm),
      grid=grid,
      in_specs=[
          pl.BlockSpec((bm, d), lambda i: (i, 0)),
          pl.BlockSpec((NC, bm, d), lambda i: (0, i, 0)),
          _full_spec(w1.shape),
          _full_spec((1, h)),
          _full_spec(w2.shape),
          _full_spec((1, h)),
      ],
      out_specs=pl.BlockSpec((bm, h), lambda i: (i, 0)),
      out_shape=jax.ShapeDtypeStruct((n_pad, h), jnp.float32),
  )(x_ext, p, w1, b1.reshape(1, -1), w2, b2.reshape(1, -1))


def _mlp2_call(hh, q, w1, b1, w2, b2, wc, bc, bm, n_real):
  d = hh.shape[1]
  n = n_real
  h = w1.shape[1]
  c = wc.shape[1]
  grid = (n // bm,)
  return pl.pallas_call(
      _mlp2_body,
      grid=grid,
      in_specs=[
          pl.BlockSpec((bm, d), lambda i: (i, 0)),
          pl.BlockSpec((NC, bm, d), lambda i: (0, i, 0)),
          _full_spec(w1.shape),
          _full_spec((1, h)),
          _full_spec(w2.shape),
          _full_spec((1, h)),
          _full_spec(wc.shape),
          _full_spec((1, c)),
      ],
      out_specs=pl.BlockSpec((bm, c), lambda i: (i, 0)),
      out_shape=jax.ShapeDtypeStruct((n, c), jnp.float32),
  )(hh, q, w1, b1.reshape(1, -1), w2, b2.reshape(1, -1),
    wc, bc.reshape(1, -1))


def kernel(x, edge_index, W11, b11, W12, b12, W21, b21, W22, b22, Wc, bc):
  n, d = x.shape
  e = edge_index.shape[1]
  nw = NC * NS
  ch = -(-e // (nw * LCH))
  ch = -(-ch // 4) * 4
  e_pad = nw * ch * LCH

  n_pad = -(-(n + 1) // (NS * 8)) * (NS * 8)

  ei = edge_index.astype(jnp.int32)
  e_tile = -(-e // nw)
  ei = jnp.pad(ei, ((0, 0), (0, nw * e_tile - e)))
  pad_per_tile = ch * LCH - e_tile
  zrows = n_pad - n
  tile_off = jnp.arange(nw, dtype=jnp.int32)[:, None] * 7
  pad_i = jnp.arange(pad_per_tile, dtype=jnp.int32)[None, :]
  pad_src = n + (pad_i + tile_off) % zrows
  pad_dst = (pad_i * 523 + tile_off * 331) % n
  if nw * e_tile == e:
    src2 = ei[0].reshape(nw, e_tile)
    dst2 = ei[1].reshape(nw, e_tile)
  else:
    real_valid = (jnp.arange(nw * e_tile, dtype=jnp.int32).reshape(nw, e_tile)
                  < e)
    src2 = jnp.where(real_valid, ei[0].reshape(nw, e_tile),
                     n + jnp.arange(e_tile, dtype=jnp.int32)[None, :] % zrows)
    dst2 = jnp.where(real_valid, ei[1].reshape(nw, e_tile), 0)
  srcp = jnp.concatenate([src2, pad_src], axis=1).reshape(nw, ch, LCH)
  dstp = jnp.concatenate([dst2, pad_dst], axis=1).reshape(nw, ch, LCH)
  zeros = jnp.zeros((n_pad // NS, d), jnp.float32)

  agg = _make_agg(n, d, n_pad, ch)
  bm = 2000

  x_ext = jnp.pad(x, ((0, n_pad - n), (0, 0)))
  p1 = agg(x_ext, srcp, dstp, zeros)
  h1 = _mlp1_call(x_ext, p1, W11, b11, W12, b12, bm, n)
  p2 = agg(h1, srcp, dstp, zeros)
  return _mlp2_call(h1, p2, W21, b21, W22, b22, Wc, bc, bm, n)


# speedup vs baseline: 3.0706x; 1.0309x over previous
import functools

import jax
import jax.numpy as jnp
from jax import lax
from jax.experimental import pallas as pl
from jax.experimental.pallas import tpu as pltpu
from jax.experimental.pallas import tpu_sc as plsc

NC = 2
NS = 16
LCH = 128


def _make_agg(n, d, n_pad, ch):
  rows_per_tile = n_pad // NS
  chp = ch // 2
  mesh = plsc.VectorSubcoreMesh(
      core_axis_name="c", subcore_axis_name="s",
      num_cores=NC, num_subcores=NS)

  @functools.partial(
      pl.kernel,
      out_type=jax.ShapeDtypeStruct((NC, n_pad, d), jnp.float32),
      mesh=mesh,
      scratch_types=[
          pltpu.VMEM((chp, LCH), jnp.int32),
          pltpu.VMEM((chp, LCH), jnp.int32),
          pltpu.VMEM((LCH, d), jnp.float32),
          pltpu.VMEM((LCH, d), jnp.float32),
          pltpu.VMEM_SHARED((n_pad, d), jnp.float32),
          pltpu.SemaphoreType.DMA,
          pltpu.SemaphoreType.DMA,
      ],
  )
  def agg_kernel(h_hbm, srcp_hbm, dstp_hbm, out_hbm,
                 src_idx, dst_idx, buf_a, buf_b, acc, sem_a, sem_b):
    c = lax.axis_index("c")
    s = lax.axis_index("s")
    wid = c * NS + s

    zv = jnp.zeros((16,), jnp.float32)

    @pl.loop(0, LCH)
    def _(r):
      for k in range(d // 16):
        buf_a[r, pl.ds(k * 16, 16)] = zv

    base = s * rows_per_tile
    full, rem = divmod(rows_per_tile, LCH)
    for b in range(full):
      pltpu.sync_copy(buf_a, acc.at[pl.ds(base + b * LCH, LCH)])
    if rem:
      pltpu.sync_copy(buf_a.at[pl.ds(0, rem)],
                      acc.at[pl.ds(base + full * LCH, rem)])
    plsc.subcore_barrier()

    def gather(j, buf, sem):
      pltpu.async_copy(h_hbm.at[src_idx.at[j]], buf, sem)

    def wait(buf, sem):
      pltpu.make_async_copy(h_hbm.at[pl.ds(0, LCH)], buf, sem).wait()

    def scatter_add(j, buf):
      pltpu.sync_copy(buf, acc.at[dst_idx.at[j]], add=True)

    @pl.loop(0, 2)
    def _(p):
      pltpu.sync_copy(srcp_hbm.at[wid, pl.ds(p * chp, chp)], src_idx)
      pltpu.sync_copy(dstp_hbm.at[wid, pl.ds(p * chp, chp)], dst_idx)

      gather(0, buf_a, sem_a)

      @pl.loop(0, chp, step=2)
      def _(g):
        gather(g + 1, buf_b, sem_b)
        wait(buf_a, sem_a)
        scatter_add(g, buf_a)

        @pl.when(g + 2 < chp)
        def _():
          gather(g + 2, buf_a, sem_a)

        wait(buf_b, sem_b)
        scatter_add(g + 1, buf_b)

    plsc.subcore_barrier()
    pltpu.sync_copy(acc.at[pl.ds(s * rows_per_tile, rows_per_tile)],
                    out_hbm.at[c, pl.ds(s * rows_per_tile, rows_per_tile)])

  return agg_kernel


def _mlp1_body(x_ref, p_ref, w1_ref, b1_ref, w2_ref, b2_ref, o_ref,
               *, n_real, bm):
  z = x_ref[...] + p_ref[0] + p_ref[1]
  t = jnp.dot(z, w1_ref[...], preferred_element_type=jnp.float32)
  t = jnp.maximum(t + b1_ref[...], 0.0)
  h = jnp.dot(t, w2_ref[...], preferred_element_type=jnp.float32)
  h = jnp.maximum(h + b2_ref[...], 0.0)
  rows = pl.program_id(0) * bm + jax.lax.broadcasted_iota(
      jnp.int32, h.shape, 0)
  o_ref[...] = jnp.where(rows < n_real, h, 0.0)


def _mlp2_body(h_ref, q_ref, w1_ref, b1_ref, w2_ref, b2_ref,
               wc_ref, bc_ref, o_ref):
  z = h_ref[...] + q_ref[0] + q_ref[1]
  t = jnp.dot(z, w1_ref[...], preferred_element_type=jnp.float32)
  t = jnp.maximum(t + b1_ref[...], 0.0)
  h2 = jnp.dot(t, w2_ref[...], preferred_element_type=jnp.float32)
  h2 = jnp.maximum(h2 + b2_ref[...], 0.0)
  o = jnp.dot(h2, wc_ref[...], preferred_element_type=jnp.float32)
  o_ref[...] = o + bc_ref[...]


def _full_spec(shape):
  return pl.BlockSpec(shape, lambda i: (0,) * len(shape))


def _mlp1_call(x_ext, p, w1, b1, w2, b2, bm, n_real):
  n_pad, d = x_ext.shape
  h = w1.shape[1]
  grid = (-(-n_pad // bm),)
  return pl.pallas_call(
      functools.partial(_mlp1_body, n_real=n_real, bm=bm),
      grid=grid,
      in_specs=[
          pl.BlockSpec((bm, d), lambda i: (i, 0)),
          pl.BlockSpec((NC, bm, d), lambda i: (0, i, 0)),
          _full_spec(w1.shape),
          _full_spec((1, h)),
          _full_spec(w2.shape),
          _full_spec((1, h)),
      ],
      out_specs=pl.BlockSpec((bm, h), lambda i: (i, 0)),
      out_shape=jax.ShapeDtypeStruct((n_pad, h), jnp.float32),
  )(x_ext, p, w1, b1.reshape(1, -1), w2, b2.reshape(1, -1))


def _mlp2_call(hh, q, w1, b1, w2, b2, wc, bc, bm, n_real):
  d = hh.shape[1]
  n = n_real
  h = w1.shape[1]
  c = wc.shape[1]
  grid = (n // bm,)
  return pl.pallas_call(
      _mlp2_body,
      grid=grid,
      in_specs=[
          pl.BlockSpec((bm, d), lambda i: (i, 0)),
          pl.BlockSpec((NC, bm, d), lambda i: (0, i, 0)),
          _full_spec(w1.shape),
          _full_spec((1, h)),
          _full_spec(w2.shape),
          _full_spec((1, h)),
          _full_spec(wc.shape),
          _full_spec((1, c)),
      ],
      out_specs=pl.BlockSpec((bm, c), lambda i: (i, 0)),
      out_shape=jax.ShapeDtypeStruct((n, c), jnp.float32),
  )(hh, q, w1, b1.reshape(1, -1), w2, b2.reshape(1, -1),
    wc, bc.reshape(1, -1))


def kernel(x, edge_index, W11, b11, W12, b12, W21, b21, W22, b22, Wc, bc):
  n, d = x.shape
  e = edge_index.shape[1]
  nw = NC * NS
  ch = -(-e // (nw * LCH))
  ch = -(-ch // 4) * 4
  e_pad = nw * ch * LCH

  n_pad = -(-(n + 1) // (NS * 8)) * (NS * 8)

  ei = edge_index.astype(jnp.int32)
  e_tile = -(-e // nw)
  ei = jnp.pad(ei, ((0, 0), (0, nw * e_tile - e)))
  pad_per_tile = ch * LCH - e_tile
  zrows = n_pad - n
  tile_off = jnp.arange(nw, dtype=jnp.int32)[:, None] * 7
  pad_i = jnp.arange(pad_per_tile, dtype=jnp.int32)[None, :]
  pad_src = n + (pad_i + tile_off) % zrows
  pad_dst = (pad_i * 523 + tile_off * 331) % n
  if nw * e_tile == e:
    src2 = ei[0].reshape(nw, e_tile)
    dst2 = ei[1].reshape(nw, e_tile)
  else:
    real_valid = (jnp.arange(nw * e_tile, dtype=jnp.int32).reshape(nw, e_tile)
                  < e)
    src2 = jnp.where(real_valid, ei[0].reshape(nw, e_tile),
                     n + jnp.arange(e_tile, dtype=jnp.int32)[None, :] % zrows)
    dst2 = jnp.where(real_valid, ei[1].reshape(nw, e_tile), 0)
  srcp = jnp.concatenate([src2, pad_src], axis=1).reshape(nw, ch, LCH)
  dstp = jnp.concatenate([dst2, pad_dst], axis=1).reshape(nw, ch, LCH)
  agg = _make_agg(n, d, n_pad, ch)
  bm = 2000

  x_ext = jnp.pad(x, ((0, n_pad - n), (0, 0)))
  p1 = agg(x_ext, srcp, dstp)
  h1 = _mlp1_call(x_ext, p1, W11, b11, W12, b12, bm, n)
  p2 = agg(h1, srcp, dstp)
  return _mlp2_call(h1, p2, W21, b21, W22, b22, Wc, bc, bm, n)


# speedup vs baseline: 3.3340x; 1.0858x over previous
import functools

import jax
import jax.numpy as jnp
from jax import lax
from jax.experimental import pallas as pl
from jax.experimental.pallas import tpu as pltpu
from jax.experimental.pallas import tpu_sc as plsc

NC = 2
NS = 16
LCH = 128


def _make_agg(n, d, n_pad, e):
  nw = NC * NS
  assert e % LCH == 0
  ch_total = e // LCH
  ch_lo = ch_total // nw
  n_hi = ch_total - ch_lo * nw
  ch_hi = ch_lo + (1 if n_hi else 0)
  chp = (ch_hi + 1) // 2
  rows_per_tile = n_pad // NS

  mesh = plsc.VectorSubcoreMesh(
      core_axis_name="c", subcore_axis_name="s",
      num_cores=NC, num_subcores=NS)

  @functools.partial(
      pl.kernel,
      out_type=jax.ShapeDtypeStruct((NC, n_pad, d), jnp.float32),
      mesh=mesh,
      scratch_types=[
          pltpu.VMEM((2, chp * LCH), jnp.int32),
          pltpu.VMEM((LCH, d), jnp.float32),
          pltpu.VMEM((LCH, d), jnp.float32),
          pltpu.VMEM_SHARED((n_pad, d), jnp.float32),
          pltpu.SemaphoreType.DMA,
          pltpu.SemaphoreType.DMA,
      ],
  )
  def agg_kernel(h_hbm, ei_hbm, out_hbm,
                 idx, buf_a, buf_b, acc, sem_a, sem_b):
    c = lax.axis_index("c")
    s = lax.axis_index("s")
    wid = c * NS + s

    zv = jnp.zeros((16,), jnp.float32)

    @pl.loop(0, LCH)
    def _(r):
      for k in range(d // 16):
        buf_a[r, pl.ds(k * 16, 16)] = zv

    zbase = s * rows_per_tile
    zfull, zrem = divmod(rows_per_tile, LCH)
    for b in range(zfull):
      pltpu.sync_copy(buf_a, acc.at[pl.ds(zbase + b * LCH, LCH)])
    if zrem:
      pltpu.sync_copy(buf_a.at[pl.ds(0, zrem)],
                      acc.at[pl.ds(zbase + zfull * LCH, zrem)])
    plsc.subcore_barrier()

    extra = wid < n_hi
    start = pl.multiple_of(
        jnp.where(extra, wid * ch_hi, n_hi * ch_hi + (wid - n_hi) * ch_lo)
        * LCH, LCH)

    def stage(phase_start, nrows):
      pltpu.sync_copy(ei_hbm.at[:, pl.ds(start + phase_start * LCH,
                                         nrows * LCH)],
                      idx.at[:, pl.ds(0, nrows * LCH)])

    def gather(j, buf, sem):
      pltpu.async_copy(h_hbm.at[idx.at[0, pl.ds(j * LCH, LCH)]], buf, sem)

    def wait(buf, sem):
      pltpu.make_async_copy(h_hbm.at[pl.ds(0, LCH)], buf, sem).wait()

    def scatter_add(j, buf):
      pltpu.sync_copy(buf, acc.at[idx.at[1, pl.ds(j * LCH, LCH)]], add=True)

    def run_phase(nch):
      gather(0, buf_a, sem_a)

      @pl.loop(0, nch, step=2)
      def _(g):
        gather(g + 1, buf_b, sem_b)
        wait(buf_a, sem_a)
        scatter_add(g, buf_a)

        @pl.when(g + 2 < nch)
        def _():
          gather(g + 2, buf_a, sem_a)

        wait(buf_b, sem_b)
        scatter_add(g + 1, buf_b)

    def run_tail(j):
      gather(j, buf_a, sem_a)
      wait(buf_a, sem_a)
      scatter_add(j, buf_a)

    assert chp % 2 == 0 and ch_lo >= chp
    stage(0, chp)
    run_phase(chp)

    r_lo, r_hi = ch_lo - chp, ch_hi - chp

    @pl.when(extra)
    def _():
      stage(chp, r_hi)
      run_phase(r_hi - (r_hi % 2))
      if r_hi % 2:
        run_tail(r_hi - 1)

    @pl.when(jnp.logical_not(extra))
    def _():
      if r_lo:
        stage(chp, r_lo)
        run_phase(r_lo - (r_lo % 2))
        if r_lo % 2:
          run_tail(r_lo - 1)

    plsc.subcore_barrier()
    pltpu.sync_copy(acc.at[pl.ds(s * rows_per_tile, rows_per_tile)],
                    out_hbm.at[c, pl.ds(s * rows_per_tile, rows_per_tile)])

  return agg_kernel


def _mlp1_body(x_ref, p_ref, w1_ref, b1_ref, w2_ref, b2_ref, o_ref):
  z = x_ref[...] + p_ref[0] + p_ref[1]
  t = jnp.dot(z, w1_ref[...], preferred_element_type=jnp.float32)
  t = jnp.maximum(t + b1_ref[...], 0.0)
  h = jnp.dot(t, w2_ref[...], preferred_element_type=jnp.float32)
  o_ref[...] = jnp.maximum(h + b2_ref[...], 0.0)


def _mlp2_body(h_ref, q_ref, w1_ref, b1_ref, w2_ref, b2_ref,
               wc_ref, bc_ref, o_ref):
  z = h_ref[...] + q_ref[0] + q_ref[1]
  t = jnp.dot(z, w1_ref[...], preferred_element_type=jnp.float32)
  t = jnp.maximum(t + b1_ref[...], 0.0)
  h2 = jnp.dot(t, w2_ref[...], preferred_element_type=jnp.float32)
  h2 = jnp.maximum(h2 + b2_ref[...], 0.0)
  o = jnp.dot(h2, wc_ref[...], preferred_element_type=jnp.float32)
  o_ref[...] = o + bc_ref[...]


def _full_spec(shape):
  return pl.BlockSpec(shape, lambda i: (0,) * len(shape))


def _mlp1_call(x, p, w1, b1, w2, b2, bm):
  n, d = x.shape
  h = w1.shape[1]
  grid = (n // bm,)
  return pl.pallas_call(
      _mlp1_body,
      grid=grid,
      in_specs=[
          pl.BlockSpec((bm, d), lambda i: (i, 0)),
          pl.BlockSpec((NC, bm, d), lambda i: (0, i, 0)),
          _full_spec(w1.shape),
          _full_spec((1, h)),
          _full_spec(w2.shape),
          _full_spec((1, h)),
      ],
      out_specs=pl.BlockSpec((bm, h), lambda i: (i, 0)),
      out_shape=jax.ShapeDtypeStruct((n, h), jnp.float32),
  )(x, p, w1, b1.reshape(1, -1), w2, b2.reshape(1, -1))


def _mlp2_call(hh, q, w1, b1, w2, b2, wc, bc, bm):
  n, d = hh.shape
  h = w1.shape[1]
  c = wc.shape[1]
  grid = (n // bm,)
  return pl.pallas_call(
      _mlp2_body,
      grid=grid,
      in_specs=[
          pl.BlockSpec((bm, d), lambda i: (i, 0)),
          pl.BlockSpec((NC, bm, d), lambda i: (0, i, 0)),
          _full_spec(w1.shape),
          _full_spec((1, h)),
          _full_spec(w2.shape),
          _full_spec((1, h)),
          _full_spec(wc.shape),
          _full_spec((1, c)),
      ],
      out_specs=pl.BlockSpec((bm, c), lambda i: (i, 0)),
      out_shape=jax.ShapeDtypeStruct((n, c), jnp.float32),
  )(hh, q, w1, b1.reshape(1, -1), w2, b2.reshape(1, -1),
    wc, bc.reshape(1, -1))


def kernel(x, edge_index, W11, b11, W12, b12, W21, b21, W22, b22, Wc, bc):
  n, d = x.shape
  e = edge_index.shape[1]
  n_pad = -(-n // (NS * 8)) * (NS * 8)

  ei = edge_index.astype(jnp.int32)
  padded = bool(e % LCH)
  if padded:
    pad = LCH - e % LCH
    ei = jnp.concatenate(
        [ei, jnp.stack([jnp.full((pad,), n, jnp.int32),
                        jnp.arange(pad, dtype=jnp.int32) * 523 % n])],
        axis=1)
    e = e + pad

  def table(t):
    if padded:
      return jnp.concatenate([t, jnp.zeros((1, d), t.dtype)], axis=0)
    return t

  agg = _make_agg(n, d, n_pad, e)
  bm = 2000

  p1 = agg(table(x), ei)
  h1 = _mlp1_call(x, p1, W11, b11, W12, b12, bm)
  p2 = agg(table(h1), ei)
  return _mlp2_call(h1, p2, W21, b21, W22, b22, Wc, bc, bm)


# speedup vs baseline: 3.3502x; 1.0048x over previous
import functools

import jax
import jax.numpy as jnp
from jax import lax
from jax.experimental import pallas as pl
from jax.experimental.pallas import tpu as pltpu
from jax.experimental.pallas import tpu_sc as plsc

NC = 2
NS = 16
LCH = 128


def _make_agg(n, d, n_pad, e):
  nw = NC * NS
  assert e % LCH == 0
  ch_total = e // LCH
  ch_lo = ch_total // nw
  n_hi = ch_total - ch_lo * nw
  ch_hi = ch_lo + (1 if n_hi else 0)
  chp = (ch_hi + 1) // 2
  rows_per_tile = n_pad // NS

  mesh = plsc.VectorSubcoreMesh(
      core_axis_name="c", subcore_axis_name="s",
      num_cores=NC, num_subcores=NS)

  @functools.partial(
      pl.kernel,
      out_type=jax.ShapeDtypeStruct((NC, n_pad, d), jnp.float32),
      mesh=mesh,
      scratch_types=[
          pltpu.VMEM((2, chp * LCH), jnp.int32),
          pltpu.VMEM((LCH, d), jnp.float32),
          pltpu.VMEM((LCH, d), jnp.float32),
          pltpu.VMEM_SHARED((n_pad, d), jnp.float32),
          pltpu.SemaphoreType.DMA,
          pltpu.SemaphoreType.DMA,
      ],
  )
  def agg_kernel(h_hbm, ei_hbm, out_hbm,
                 idx, buf_a, buf_b, acc, sem_a, sem_b):
    c = lax.axis_index("c")
    s = lax.axis_index("s")
    wid = c * NS + s

    zv = jnp.zeros((16,), jnp.float32)

    @pl.loop(0, LCH)
    def _(r):
      for k in range(d // 16):
        buf_a[r, pl.ds(k * 16, 16)] = zv

    zbase = s * rows_per_tile
    zfull, zrem = divmod(rows_per_tile, LCH)
    for b in range(zfull):
      pltpu.sync_copy(buf_a, acc.at[pl.ds(zbase + b * LCH, LCH)])
    if zrem:
      pltpu.sync_copy(buf_a.at[pl.ds(0, zrem)],
                      acc.at[pl.ds(zbase + zfull * LCH, zrem)])
    plsc.subcore_barrier()

    r_c = n_hi // NC + jnp.where(c < n_hi % NC, 1, 0)
    core_base = c * NS * ch_lo + c * (n_hi // NC) + jnp.minimum(c, n_hi % NC)
    extra = s < r_c
    start = pl.multiple_of(
        (core_base
         + jnp.where(extra, s * ch_hi, r_c * ch_hi + (s - r_c) * ch_lo))
        * LCH, LCH)

    def stage(phase_start, nrows):
      pltpu.sync_copy(ei_hbm.at[:, pl.ds(start + phase_start * LCH,
                                         nrows * LCH)],
                      idx.at[:, pl.ds(0, nrows * LCH)])

    def gather(j, buf, sem):
      pltpu.async_copy(h_hbm.at[idx.at[0, pl.ds(j * LCH, LCH)]], buf, sem)

    def wait(buf, sem):
      pltpu.make_async_copy(h_hbm.at[pl.ds(0, LCH)], buf, sem).wait()

    def scatter_add(j, buf):
      pltpu.sync_copy(buf, acc.at[idx.at[1, pl.ds(j * LCH, LCH)]], add=True)

    def run_phase(nch):
      gather(0, buf_a, sem_a)

      @pl.loop(0, nch, step=2)
      def _(g):
        gather(g + 1, buf_b, sem_b)
        wait(buf_a, sem_a)
        scatter_add(g, buf_a)

        @pl.when(g + 2 < nch)
        def _():
          gather(g + 2, buf_a, sem_a)

        wait(buf_b, sem_b)
        scatter_add(g + 1, buf_b)

    def run_tail(j):
      gather(j, buf_a, sem_a)
      wait(buf_a, sem_a)
      scatter_add(j, buf_a)

    assert chp % 2 == 0 and ch_lo >= chp
    stage(0, chp)
    run_phase(chp)

    r_lo, r_hi = ch_lo - chp, ch_hi - chp

    @pl.when(extra)
    def _():
      stage(chp, r_hi)
      run_phase(r_hi - (r_hi % 2))
      if r_hi % 2:
        run_tail(r_hi - 1)

    @pl.when(jnp.logical_not(extra))
    def _():
      if r_lo:
        stage(chp, r_lo)
        run_phase(r_lo - (r_lo % 2))
        if r_lo % 2:
          run_tail(r_lo - 1)

    plsc.subcore_barrier()
    pltpu.sync_copy(acc.at[pl.ds(s * rows_per_tile, rows_per_tile)],
                    out_hbm.at[c, pl.ds(s * rows_per_tile, rows_per_tile)])

  return agg_kernel


def _mlp1_body(x_ref, p_ref, w1_ref, b1_ref, w2_ref, b2_ref, o_ref):
  z = x_ref[...] + p_ref[0] + p_ref[1]
  t = jnp.dot(z, w1_ref[...], preferred_element_type=jnp.float32)
  t = jnp.maximum(t + b1_ref[...], 0.0)
  h = jnp.dot(t, w2_ref[...], preferred_element_type=jnp.float32)
  o_ref[...] = jnp.maximum(h + b2_ref[...], 0.0)


def _mlp2_body(h_ref, q_ref, w1_ref, b1_ref, w2_ref, b2_ref,
               wc_ref, bc_ref, o_ref):
  z = h_ref[...] + q_ref[0] + q_ref[1]
  t = jnp.dot(z, w1_ref[...], preferred_element_type=jnp.float32)
  t = jnp.maximum(t + b1_ref[...], 0.0)
  h2 = jnp.dot(t, w2_ref[...], preferred_element_type=jnp.float32)
  h2 = jnp.maximum(h2 + b2_ref[...], 0.0)
  o = jnp.dot(h2, wc_ref[...], preferred_element_type=jnp.float32)
  o_ref[...] = o + bc_ref[...]


def _full_spec(shape):
  return pl.BlockSpec(shape, lambda i: (0,) * len(shape))


def _mlp1_call(x, p, w1, b1, w2, b2, bm):
  n, d = x.shape
  h = w1.shape[1]
  grid = (n // bm,)
  return pl.pallas_call(
      _mlp1_body,
      grid=grid,
      in_specs=[
          pl.BlockSpec((bm, d), lambda i: (i, 0)),
          pl.BlockSpec((NC, bm, d), lambda i: (0, i, 0)),
          _full_spec(w1.shape),
          _full_spec((1, h)),
          _full_spec(w2.shape),
          _full_spec((1, h)),
      ],
      out_specs=pl.BlockSpec((bm, h), lambda i: (i, 0)),
      out_shape=jax.ShapeDtypeStruct((n, h), jnp.float32),
  )(x, p, w1, b1.reshape(1, -1), w2, b2.reshape(1, -1))


def _mlp2_call(hh, q, w1, b1, w2, b2, wc, bc, bm):
  n, d = hh.shape
  h = w1.shape[1]
  c = wc.shape[1]
  grid = (n // bm,)
  return pl.pallas_call(
      _mlp2_body,
      grid=grid,
      in_specs=[
          pl.BlockSpec((bm, d), lambda i: (i, 0)),
          pl.BlockSpec((NC, bm, d), lambda i: (0, i, 0)),
          _full_spec(w1.shape),
          _full_spec((1, h)),
          _full_spec(w2.shape),
          _full_spec((1, h)),
          _full_spec(wc.shape),
          _full_spec((1, c)),
      ],
      out_specs=pl.BlockSpec((bm, c), lambda i: (i, 0)),
      out_shape=jax.ShapeDtypeStruct((n, c), jnp.float32),
  )(hh, q, w1, b1.reshape(1, -1), w2, b2.reshape(1, -1),
    wc, bc.reshape(1, -1))


def kernel(x, edge_index, W11, b11, W12, b12, W21, b21, W22, b22, Wc, bc):
  n, d = x.shape
  e = edge_index.shape[1]
  n_pad = -(-n // (NS * 8)) * (NS * 8)

  ei = edge_index.astype(jnp.int32)
  padded = bool(e % LCH)
  if padded:
    pad = LCH - e % LCH
    ei = jnp.concatenate(
        [ei, jnp.stack([jnp.full((pad,), n, jnp.int32),
                        jnp.arange(pad, dtype=jnp.int32) * 523 % n])],
        axis=1)
    e = e + pad

  def table(t):
    if padded:
      return jnp.concatenate([t, jnp.zeros((1, d), t.dtype)], axis=0)
    return t

  agg = _make_agg(n, d, n_pad, e)
  bm = 2000

  p1 = agg(table(x), ei)
  h1 = _mlp1_call(x, p1, W11, b11, W12, b12, bm)
  p2 = agg(table(h1), ei)
  return _mlp2_call(h1, p2, W21, b21, W22, b22, Wc, bc, bm)


# speedup vs baseline: 3.3619x; 1.0035x over previous
import functools

import jax
import jax.numpy as jnp
from jax import lax
from jax.experimental import pallas as pl
from jax.experimental.pallas import tpu as pltpu
from jax.experimental.pallas import tpu_sc as plsc

NC = 2
NS = 16
LCH = 128


def _make_agg(n, d, n_pad, e):
  nw = NC * NS
  assert e % LCH == 0
  ch_total = e // LCH
  ch_lo = ch_total // nw
  n_hi = ch_total - ch_lo * nw
  ch_hi = ch_lo + (1 if n_hi else 0)
  chp = (ch_hi + 1) // 2
  rows_per_tile = n_pad // NS

  mesh = plsc.VectorSubcoreMesh(
      core_axis_name="c", subcore_axis_name="s",
      num_cores=NC, num_subcores=NS)

  @functools.partial(
      pl.kernel,
      out_type=jax.ShapeDtypeStruct((NC, n_pad, d), jnp.float32),
      mesh=mesh,
      scratch_types=[
          pltpu.VMEM((2, chp * LCH), jnp.int32),
          pltpu.VMEM((LCH, d), jnp.float32),
          pltpu.VMEM((LCH, d), jnp.float32),
          pltpu.VMEM_SHARED((n_pad, d), jnp.float32),
          pltpu.SemaphoreType.DMA,
          pltpu.SemaphoreType.DMA,
      ],
  )
  def agg_kernel(h_hbm, ei_hbm, out_hbm,
                 idx, buf_a, buf_b, acc, sem_a, sem_b):
    c = lax.axis_index("c")
    s = lax.axis_index("s")
    wid = c * NS + s

    zv = jnp.zeros((16,), jnp.float32)

    @pl.loop(0, LCH)
    def _(r):
      for k in range(d // 16):
        buf_a[r, pl.ds(k * 16, 16)] = zv

    zbase = s * rows_per_tile
    zfull, zrem = divmod(rows_per_tile, LCH)
    for b in range(zfull):
      pltpu.sync_copy(buf_a, acc.at[pl.ds(zbase + b * LCH, LCH)])
    if zrem:
      pltpu.sync_copy(buf_a.at[pl.ds(0, zrem)],
                      acc.at[pl.ds(zbase + zfull * LCH, zrem)])
    plsc.subcore_barrier()

    r_c = n_hi // NC + jnp.where(c < n_hi % NC, 1, 0)
    core_base = c * NS * ch_lo + c * (n_hi // NC) + jnp.minimum(c, n_hi % NC)
    extra = s < r_c
    start = pl.multiple_of(
        (core_base
         + jnp.where(extra, s * ch_hi, r_c * ch_hi + (s - r_c) * ch_lo))
        * LCH, LCH)

    def stage(phase_start, nrows):
      pltpu.sync_copy(ei_hbm.at[:, pl.ds(start + phase_start * LCH,
                                         nrows * LCH)],
                      idx.at[:, pl.ds(0, nrows * LCH)])

    def gather(j, buf, sem):
      half = LCH // 2
      pltpu.async_copy(h_hbm.at[idx.at[0, pl.ds(j * LCH, half)]],
                       buf.at[pl.ds(0, half)], sem)
      pltpu.async_copy(h_hbm.at[idx.at[0, pl.ds(j * LCH + half, half)]],
                       buf.at[pl.ds(half, half)], sem)

    def wait(buf, sem):
      pltpu.make_async_copy(h_hbm.at[pl.ds(0, LCH)], buf, sem).wait()

    def scatter_add(j, buf):
      pltpu.sync_copy(buf, acc.at[idx.at[1, pl.ds(j * LCH, LCH)]], add=True)

    def run_phase(nch):
      gather(0, buf_a, sem_a)

      @pl.loop(0, nch, step=2)
      def _(g):
        gather(g + 1, buf_b, sem_b)
        wait(buf_a, sem_a)
        scatter_add(g, buf_a)

        @pl.when(g + 2 < nch)
        def _():
          gather(g + 2, buf_a, sem_a)

        wait(buf_b, sem_b)
        scatter_add(g + 1, buf_b)

    def run_tail(j):
      gather(j, buf_a, sem_a)
      wait(buf_a, sem_a)
      scatter_add(j, buf_a)

    assert chp % 2 == 0 and ch_lo >= chp
    stage(0, chp)
    run_phase(chp)

    r_lo, r_hi = ch_lo - chp, ch_hi - chp

    @pl.when(extra)
    def _():
      stage(chp, r_hi)
      run_phase(r_hi - (r_hi % 2))
      if r_hi % 2:
        run_tail(r_hi - 1)

    @pl.when(jnp.logical_not(extra))
    def _():
      if r_lo:
        stage(chp, r_lo)
        run_phase(r_lo - (r_lo % 2))
        if r_lo % 2:
          run_tail(r_lo - 1)

    plsc.subcore_barrier()
    pltpu.sync_copy(acc.at[pl.ds(s * rows_per_tile, rows_per_tile)],
                    out_hbm.at[c, pl.ds(s * rows_per_tile, rows_per_tile)])

  return agg_kernel


def _mlp1_body(x_ref, p_ref, w1_ref, b1_ref, w2_ref, b2_ref, o_ref):
  z = x_ref[...] + p_ref[0] + p_ref[1]
  t = jnp.dot(z, w1_ref[...], preferred_element_type=jnp.float32)
  t = jnp.maximum(t + b1_ref[...], 0.0)
  h = jnp.dot(t, w2_ref[...], preferred_element_type=jnp.float32)
  o_ref[...] = jnp.maximum(h + b2_ref[...], 0.0)


def _mlp2_body(h_ref, q_ref, w1_ref, b1_ref, w2_ref, b2_ref,
               wc_ref, bc_ref, o_ref):
  z = h_ref[...] + q_ref[0] + q_ref[1]
  t = jnp.dot(z, w1_ref[...], preferred_element_type=jnp.float32)
  t = jnp.maximum(t + b1_ref[...], 0.0)
  h2 = jnp.dot(t, w2_ref[...], preferred_element_type=jnp.float32)
  h2 = jnp.maximum(h2 + b2_ref[...], 0.0)
  o = jnp.dot(h2, wc_ref[...], preferred_element_type=jnp.float32)
  o_ref[...] = o + bc_ref[...]


def _full_spec(shape):
  return pl.BlockSpec(shape, lambda i: (0,) * len(shape))


def _mlp1_call(x, p, w1, b1, w2, b2, bm):
  n, d = x.shape
  h = w1.shape[1]
  grid = (n // bm,)
  return pl.pallas_call(
      _mlp1_body,
      grid=grid,
      in_specs=[
          pl.BlockSpec((bm, d), lambda i: (i, 0)),
          pl.BlockSpec((NC, bm, d), lambda i: (0, i, 0)),
          _full_spec(w1.shape),
          _full_spec((1, h)),
          _full_spec(w2.shape),
          _full_spec((1, h)),
      ],
      out_specs=pl.BlockSpec((bm, h), lambda i: (i, 0)),
      out_shape=jax.ShapeDtypeStruct((n, h), jnp.float32),
  )(x, p, w1, b1.reshape(1, -1), w2, b2.reshape(1, -1))


def _mlp2_call(hh, q, w1, b1, w2, b2, wc, bc, bm):
  n, d = hh.shape
  h = w1.shape[1]
  c = wc.shape[1]
  grid = (n // bm,)
  return pl.pallas_call(
      _mlp2_body,
      grid=grid,
      in_specs=[
          pl.BlockSpec((bm, d), lambda i: (i, 0)),
          pl.BlockSpec((NC, bm, d), lambda i: (0, i, 0)),
          _full_spec(w1.shape),
          _full_spec((1, h)),
          _full_spec(w2.shape),
          _full_spec((1, h)),
          _full_spec(wc.shape),
          _full_spec((1, c)),
      ],
      out_specs=pl.BlockSpec((bm, c), lambda i: (i, 0)),
      out_shape=jax.ShapeDtypeStruct((n, c), jnp.float32),
  )(hh, q, w1, b1.reshape(1, -1), w2, b2.reshape(1, -1),
    wc, bc.reshape(1, -1))


def kernel(x, edge_index, W11, b11, W12, b12, W21, b21, W22, b22, Wc, bc):
  n, d = x.shape
  e = edge_index.shape[1]
  n_pad = -(-n // (NS * 8)) * (NS * 8)

  ei = edge_index.astype(jnp.int32)
  padded = bool(e % LCH)
  if padded:
    pad = LCH - e % LCH
    ei = jnp.concatenate(
        [ei, jnp.stack([jnp.full((pad,), n, jnp.int32),
                        jnp.arange(pad, dtype=jnp.int32) * 523 % n])],
        axis=1)
    e = e + pad

  def table(t):
    if padded:
      return jnp.concatenate([t, jnp.zeros((1, d), t.dtype)], axis=0)
    return t

  agg = _make_agg(n, d, n_pad, e)
  bm = 2000

  p1 = agg(table(x), ei)
  h1 = _mlp1_call(x, p1, W11, b11, W12, b12, bm)
  p2 = agg(table(h1), ei)
  return _mlp2_call(h1, p2, W21, b21, W22, b22, Wc, bc, bm)


# speedup vs baseline: 3.3909x; 1.0086x over previous
import functools

import jax
import jax.numpy as jnp
from jax import lax
from jax.experimental import pallas as pl
from jax.experimental.pallas import tpu as pltpu
from jax.experimental.pallas import tpu_sc as plsc

NC = 2
NS = 16
LCH = 128


def _make_agg(n, d, n_pad, e):
  nw = NC * NS
  assert e % LCH == 0
  ch_total = e // LCH
  ch_lo = ch_total // nw
  n_hi = ch_total - ch_lo * nw
  ch_hi = ch_lo + (1 if n_hi else 0)
  chp = (ch_hi + 1) // 2
  rows_per_tile = n_pad // NS

  mesh = plsc.VectorSubcoreMesh(
      core_axis_name="c", subcore_axis_name="s",
      num_cores=NC, num_subcores=NS)

  @functools.partial(
      pl.kernel,
      out_type=jax.ShapeDtypeStruct((NC, n_pad, d), jnp.float32),
      mesh=mesh,
      scratch_types=[
          pltpu.VMEM((2, chp * LCH), jnp.int32),
          pltpu.VMEM((LCH, d), jnp.float32),
          pltpu.VMEM((LCH, d), jnp.float32),
          pltpu.VMEM_SHARED((n_pad, d), jnp.float32),
          pltpu.SemaphoreType.DMA,
          pltpu.SemaphoreType.DMA,
      ],
  )
  def agg_kernel(h_hbm, ei_hbm, out_hbm,
                 idx, buf_a, buf_b, acc, sem_a, sem_b):
    c = lax.axis_index("c")
    s = lax.axis_index("s")
    wid = c * NS + s

    zv = jnp.zeros((16,), jnp.float32)

    @pl.loop(0, LCH)
    def _(r):
      for k in range(d // 16):
        buf_a[r, pl.ds(k * 16, 16)] = zv

    zbase = s * rows_per_tile
    zfull, zrem = divmod(rows_per_tile, LCH)
    for b in range(zfull):
      pltpu.sync_copy(buf_a, acc.at[pl.ds(zbase + b * LCH, LCH)])
    if zrem:
      pltpu.sync_copy(buf_a.at[pl.ds(0, zrem)],
                      acc.at[pl.ds(zbase + zfull * LCH, zrem)])
    plsc.subcore_barrier()

    r_c = n_hi // NC + jnp.where(c < n_hi % NC, 1, 0)
    core_base = c * NS * ch_lo + c * (n_hi // NC) + jnp.minimum(c, n_hi % NC)
    extra = s < r_c
    start = pl.multiple_of(
        (core_base
         + jnp.where(extra, s * ch_hi, r_c * ch_hi + (s - r_c) * ch_lo))
        * LCH, LCH)

    def stage(phase_start, nrows):
      pltpu.sync_copy(ei_hbm.at[:, pl.ds(start + phase_start * LCH,
                                         nrows * LCH)],
                      idx.at[:, pl.ds(0, nrows * LCH)])

    def gather(j, buf, sem):
      half = LCH // 2
      pltpu.async_copy(h_hbm.at[idx.at[0, pl.ds(j * LCH, half)]],
                       buf.at[pl.ds(0, half)], sem)
      pltpu.async_copy(h_hbm.at[idx.at[0, pl.ds(j * LCH + half, half)]],
                       buf.at[pl.ds(half, half)], sem)

    def wait(buf, sem):
      pltpu.make_async_copy(h_hbm.at[pl.ds(0, LCH)], buf, sem).wait()

    def scatter_add(j, buf):
      pltpu.sync_copy(buf, acc.at[idx.at[1, pl.ds(j * LCH, LCH)]], add=True)

    def run_phase(nch):
      gather(0, buf_a, sem_a)

      @pl.loop(0, nch, step=2)
      def _(g):
        gather(g + 1, buf_b, sem_b)
        wait(buf_a, sem_a)
        scatter_add(g, buf_a)

        @pl.when(g + 2 < nch)
        def _():
          gather(g + 2, buf_a, sem_a)

        wait(buf_b, sem_b)
        scatter_add(g + 1, buf_b)

    def run_tail(j):
      gather(j, buf_a, sem_a)
      wait(buf_a, sem_a)
      scatter_add(j, buf_a)

    assert chp % 2 == 0 and ch_lo >= chp
    stage(0, chp)
    run_phase(chp)

    r_lo, r_hi = ch_lo - chp, ch_hi - chp

    @pl.when(extra)
    def _():
      stage(chp, r_hi)
      run_phase(r_hi - (r_hi % 2))
      if r_hi % 2:
        run_tail(r_hi - 1)

    @pl.when(jnp.logical_not(extra))
    def _():
      if r_lo:
        stage(chp, r_lo)
        run_phase(r_lo - (r_lo % 2))
        if r_lo % 2:
          run_tail(r_lo - 1)

    plsc.subcore_barrier()
    pltpu.sync_copy(acc.at[pl.ds(s * rows_per_tile, rows_per_tile)],
                    out_hbm.at[c, pl.ds(s * rows_per_tile, rows_per_tile)])

  return agg_kernel


def _mlp1_body(x_ref, p_ref, w1_ref, b1_ref, w2_ref, b2_ref, o_ref):
  z = x_ref[...] + p_ref[0] + p_ref[1]
  t = jnp.dot(z, w1_ref[...], preferred_element_type=jnp.float32)
  t = jnp.maximum(t + b1_ref[...], 0.0)
  h = jnp.dot(t, w2_ref[...], preferred_element_type=jnp.float32)
  o_ref[...] = jnp.maximum(h + b2_ref[...], 0.0)


def _mlp2_body(h_ref, q_ref, w1_ref, b1_ref, w2_ref, b2_ref,
               wc_ref, bc_ref, o_ref):
  z = h_ref[...] + q_ref[0] + q_ref[1]
  t = jnp.dot(z, w1_ref[...], preferred_element_type=jnp.float32)
  t = jnp.maximum(t + b1_ref[...], 0.0)
  h2 = jnp.dot(t, w2_ref[...], preferred_element_type=jnp.float32)
  h2 = jnp.maximum(h2 + b2_ref[...], 0.0)
  o = jnp.dot(h2, wc_ref[...], preferred_element_type=jnp.float32)
  o_ref[...] = o + bc_ref[...]


def _full_spec(shape):
  return pl.BlockSpec(shape, lambda i: (0,) * len(shape))


def _mlp1_call(x, p, w1, b1, w2, b2, bm):
  n, d = x.shape
  h = w1.shape[1]
  grid = (n // bm,)
  return pl.pallas_call(
      _mlp1_body,
      grid=grid,
      in_specs=[
          pl.BlockSpec((bm, d), lambda i: (i, 0)),
          pl.BlockSpec((NC, bm, d), lambda i: (0, i, 0)),
          _full_spec(w1.shape),
          _full_spec((1, h)),
          _full_spec(w2.shape),
          _full_spec((1, h)),
      ],
      out_specs=pl.BlockSpec((bm, h), lambda i: (i, 0)),
      out_shape=jax.ShapeDtypeStruct((n, h), jnp.float32),
  )(x, p, w1, b1.reshape(1, -1), w2, b2.reshape(1, -1))


def _mlp2_call(hh, q, w1, b1, w2, b2, wc, bc, bm):
  n, d = hh.shape
  h = w1.shape[1]
  c = wc.shape[1]
  grid = (n // bm,)
  return pl.pallas_call(
      _mlp2_body,
      grid=grid,
      in_specs=[
          pl.BlockSpec((bm, d), lambda i: (i, 0)),
          pl.BlockSpec((NC, bm, d), lambda i: (0, i, 0)),
          _full_spec(w1.shape),
          _full_spec((1, h)),
          _full_spec(w2.shape),
          _full_spec((1, h)),
          _full_spec(wc.shape),
          _full_spec((1, c)),
      ],
      out_specs=pl.BlockSpec((bm, c), lambda i: (i, 0)),
      out_shape=jax.ShapeDtypeStruct((n, c), jnp.float32),
  )(hh, q, w1, b1.reshape(1, -1), w2, b2.reshape(1, -1),
    wc, bc.reshape(1, -1))


def kernel(x, edge_index, W11, b11, W12, b12, W21, b21, W22, b22, Wc, bc):
  n, d = x.shape
  e = edge_index.shape[1]
  n_pad = -(-n // (NS * 8)) * (NS * 8)

  ei = edge_index.astype(jnp.int32)
  padded = bool(e % LCH)
  if padded:
    pad = LCH - e % LCH
    ei = jnp.concatenate(
        [ei, jnp.stack([jnp.full((pad,), n, jnp.int32),
                        jnp.arange(pad, dtype=jnp.int32) * 523 % n])],
        axis=1)
    e = e + pad

  def table(t):
    if padded:
      return jnp.concatenate([t, jnp.zeros((1, d), t.dtype)], axis=0)
    return t

  agg = _make_agg(n, d, n_pad, e)
  bm = 5000 if n % 5000 == 0 else 2000

  p1 = agg(table(x), ei)
  h1 = _mlp1_call(x, p1, W11, b11, W12, b12, bm)
  p2 = agg(table(h1), ei)
  return _mlp2_call(h1, p2, W21, b21, W22, b22, Wc, bc, bm)
